# baseline (speedup 1.0000x reference)
import functools

import jax
import jax.numpy as jnp
from jax import lax
from jax.experimental import pallas as pl
from jax.experimental.pallas import tpu as pltpu
from jax.experimental.pallas import tpu_sc as plsc

_N = 10000
_NP = 10240
_GP = 80
_LN = 128
_KNN = 16
_C = 64
_BIG = 3.0e38


def _ln(x, g, b):
    m = x.mean(-1, keepdims=True)
    v = ((x - m) ** 2).mean(-1, keepdims=True)
    return (x - m) / jnp.sqrt(v + 1e-5) * g + b


def _gelu(x):
    return x * 0.5 * (1.0 + lax.erf(x * (2.0 ** -0.5)))


_R = 64
_NPASS = 3


def _knn_body(pxr, pyr, pzr, pxa, pya, pza, idx_out, wv_out, di_out):
    base = pl.program_id(0) * _R
    px = pxr[...]
    py = pyr[...]
    pz = pzr[...]

    def gstep(g, carry):
        m1, m2, m3 = carry
        dx = px - pxa[g, :][None, :]
        dy = py - pya[g, :][None, :]
        dz = pz - pza[g, :][None, :]
        d = dx * dx + dy * dy + dz * dz
        key = lax.bitcast_convert_type(
            (lax.bitcast_convert_type(d + 1.0, jnp.int32) & -128) | g,
            jnp.float32)
        lo1 = jnp.minimum(m1, key)
        hi1 = jnp.maximum(m1, key)
        lo2 = jnp.minimum(m2, hi1)
        hi2 = jnp.maximum(m2, hi1)
        return lo1, lo2, jnp.minimum(m3, hi2)

    full = jnp.full((_R, _LN), _BIG, jnp.float32)
    ms = lax.fori_loop(0, _GP, gstep, (full, full, full))

    pool = jnp.concatenate(ms, axis=1)
    slot = lax.broadcasted_iota(jnp.int32, (_R, _NPASS * _LN), 1)
    pool_bits = lax.bitcast_convert_type(pool, jnp.int32)
    col = (pool_bits & 127) * _LN + (slot & 127)
    row = base + lax.broadcasted_iota(jnp.int32, (_R, _NPASS * _LN), 0)
    pool = jnp.where(col == row, _BIG, pool)

    idx_cols, wv_cols = [], []
    for _ in range(_KNN):
        m = jnp.min(pool, axis=1, keepdims=True)
        ic = jnp.min(jnp.where(pool == m, col, _NP), axis=1, keepdims=True)
        idx_cols.append(ic)
        dm = lax.bitcast_convert_type(
            lax.bitcast_convert_type(m, jnp.int32) & -128, jnp.float32) - 1.0
        wv_cols.append(jnp.exp(-dm))
        pool = jnp.where(col == ic, _BIG, pool)

    wv = jnp.concatenate(wv_cols, axis=1)
    idx_out[...] = jnp.concatenate(idx_cols, axis=1)
    wv_out[...] = wv
    deg = 1.0 + jnp.sum(wv, axis=1, keepdims=True)
    di_out[...] = jnp.minimum(lax.rsqrt(deg), 1e4)


def _knn(pos):
    pos = pos * jnp.array([[50000000.0 ** -0.5, 50000000.0 ** -0.5,
                            180000.0 ** -0.5]], jnp.float32)
    pxr = pos[:, 0:1]
    pyr = pos[:, 1:2]
    pzr = pos[:, 2:3]
    pxa = pos[:, 0].reshape(_GP, _LN)
    pya = pos[:, 1].reshape(_GP, _LN)
    pza = pos[:, 2].reshape(_GP, _LN)
    rspec = pl.BlockSpec((_R, 1), lambda i: (i, 0))
    aspec = pl.BlockSpec((_GP, _LN), lambda i: (0, 0))
    return pl.pallas_call(
        _knn_body,
        grid=(_NP // _R,),
        in_specs=[rspec, rspec, rspec, aspec, aspec, aspec],
        out_specs=[pl.BlockSpec((_R, _KNN), lambda i: (i, 0)),
                   pl.BlockSpec((_R, _KNN), lambda i: (i, 0)),
                   pl.BlockSpec((_R, 1), lambda i: (i, 0))],
        out_shape=[jax.ShapeDtypeStruct((_NP, _KNN), jnp.int32),
                   jax.ShapeDtypeStruct((_NP, _KNN), jnp.float32),
                   jax.ShapeDtypeStruct((_NP, 1), jnp.float32)],
        compiler_params=pltpu.CompilerParams(
            dimension_semantics=("arbitrary",)),
    )(pxr, pyr, pzr, pxa, pya, pza)


_B = 512


def _embed_body(xs5, xs6, xs7, xs8, xs9, xi5, xi6, xi7, xi8, xi9, di,
                Ws, bs, gs, bes, Wi, bi, gi, bei,
                cwt1, cb1, bng1, bnb1, cwt2, cb2, bng2, bnb2,
                h_out, hp_out):
    xs = [xs5, xs6, xs7, xs8, xs9]
    xi = [xi5, xi6, xi7, xi8, xi9]
    e = []
    for t in range(5):
        es = jnp.dot(xs[t][...], Ws[...], preferred_element_type=jnp.float32)
        ei = jnp.dot(xi[t][...], Wi[...], preferred_element_type=jnp.float32)
        es = _gelu(_ln(es + bs[...], gs[...], bes[...]))
        ei = _gelu(_ln(ei + bi[...], gi[...], bei[...]))
        e.append(es + ei)
    bnscale = 1.0 / jnp.sqrt(1.0 + 1e-5)
    o1 = []
    for t in range(2, 5):
        c = (jnp.dot(e[t - 2][...], cwt1[0], preferred_element_type=jnp.float32)
             + jnp.dot(e[t - 1][...], cwt1[1], preferred_element_type=jnp.float32)
             + jnp.dot(e[t][...], cwt1[2], preferred_element_type=jnp.float32)
             + cb1[...])
        o1.append(_gelu(c * bnscale * bng1[...] + bnb1[...]))
    c2 = (jnp.dot(o1[0], cwt2[0], preferred_element_type=jnp.float32)
          + jnp.dot(o1[1], cwt2[1], preferred_element_type=jnp.float32)
          + jnp.dot(o1[2], cwt2[2], preferred_element_type=jnp.float32)
          + cb2[...])
    o2 = _gelu(c2 * bnscale * bng2[...] + bnb2[...])
    h = o2 + e[4]
    h_out[...] = h
    hp_out[...] = h * di[...]


def _embed(xp, di, p):
    xs = [xp[:, t, :9] for t in range(5, 10)]
    xi = [xp[:, t, 9:] for t in range(5, 10)]
    cwt1 = jnp.transpose(p['cw1'], (2, 1, 0))
    cwt2 = jnp.transpose(p['cw2'], (2, 1, 0))
    r1 = lambda a: a.reshape(1, -1)
    bspec = lambda w: pl.BlockSpec((_B, w), lambda i: (i, 0))
    fspec = lambda *s: pl.BlockSpec(s, lambda i: tuple(0 for _ in s))
    args = (xs + xi
            + [di, p['Ws'], r1(p['bs']), r1(p['gs']), r1(p['bes']),
               p['Wi'], r1(p['bi']), r1(p['gi']), r1(p['bei']),
               cwt1, r1(p['cb1']), r1(p['bng1']), r1(p['bnb1']),
               cwt2, r1(p['cb2']), r1(p['bng2']), r1(p['bnb2'])])
    specs = ([bspec(9)] * 5 + [bspec(19)] * 5
             + [bspec(1), fspec(9, _C), fspec(1, _C), fspec(1, _C), fspec(1, _C),
                fspec(19, _C), fspec(1, _C), fspec(1, _C), fspec(1, _C),
                fspec(3, _C, _C), fspec(1, _C), fspec(1, _C), fspec(1, _C),
                fspec(3, _C, _C), fspec(1, _C), fspec(1, _C), fspec(1, _C)])
    return pl.pallas_call(
        _embed_body,
        grid=(_NP // _B,),
        in_specs=specs,
        out_specs=[bspec(_C), bspec(_C)],
        out_shape=[jax.ShapeDtypeStruct((_NP, _C), jnp.float32),
                   jax.ShapeDtypeStruct((_NP, _C), jnp.float32)],
        compiler_params=pltpu.CompilerParams(
            dimension_semantics=("arbitrary",)),
    )(*args)


_NW = 32
_EP = _NP * _KNN
_BPW = _EP // _NW
_CHUNK = 128
_NCH = _BPW // _CHUNK


_NBUF = 4
_RPC = _CHUNK // _KNN


def _sc_gather_body(table_hbm, idx_hbm, wv_hbm, out_hbm, idx_v, wv_v, acc_v,
                    *bufs_sems):
    bufs = bufs_sems[:_NBUF]
    sems = bufs_sems[_NBUF:]
    wid = lax.axis_index("s") * 2 + lax.axis_index("c")
    rbase = wid * (_BPW // _KNN)
    pltpu.sync_copy(idx_hbm.at[wid], idx_v)
    pltpu.sync_copy(wv_hbm.at[wid], wv_v)
    for b in range(_NBUF):
        pltpu.async_copy(table_hbm.at[idx_v.at[b]], bufs[b], sems[b])

    def _reduce_chunk(j, b):
        def rbody(r, carry):
            wvr = wv_v[j, pl.ds(r * _KNN, _KNN)]
            nq = _C // 16
            accs = [jnp.zeros((16,), jnp.float32)] * nq
            for k in range(_KNN):
                w = wvr[k]
                for q in range(nq):
                    accs[q] = accs[q] + w * bufs[b][r * _KNN + k,
                                                    pl.ds(q * 16, 16)]
            for q in range(nq):
                acc_v[r, pl.ds(q * 16, 16)] = accs[q]
            return carry

        lax.fori_loop(0, _RPC, rbody, 0)
        pltpu.sync_copy(acc_v, out_hbm.at[pl.ds(rbase + j * _RPC, _RPC)])

    def body(i, carry):
        for b in range(_NBUF):
            j = _NBUF * i + b
            pltpu.make_async_copy(
                table_hbm.at[idx_v.at[j]], bufs[b], sems[b]).wait()
            _reduce_chunk(j, b)
            pltpu.async_copy(table_hbm.at[idx_v.at[j + _NBUF]], bufs[b],
                             sems[b])
        return carry

    lax.fori_loop(0, _NCH // _NBUF - 1, body, 0)
    for b in range(_NBUF):
        j = _NCH - _NBUF + b
        pltpu.make_async_copy(
            table_hbm.at[idx_v.at[j]], bufs[b], sems[b]).wait()
        _reduce_chunk(j, b)


def _gather_reduce(table, idx3, wv3):
    mesh = plsc.VectorSubcoreMesh(core_axis_name="c", subcore_axis_name="s")
    f = functools.partial(
        pl.kernel,
        mesh=mesh,
        out_type=jax.ShapeDtypeStruct((_NP, _C), jnp.float32),
        scratch_types=(
            [pltpu.VMEM((_NCH, _CHUNK), jnp.int32),
             pltpu.VMEM((_NCH, _CHUNK), jnp.float32),
             pltpu.VMEM((_RPC, _C), jnp.float32)]
            + [pltpu.VMEM((_CHUNK, _C), jnp.float32)] * _NBUF
            + [pltpu.SemaphoreType.DMA] * _NBUF
        ),
        compiler_params=pltpu.CompilerParams(use_tc_tiling_on_sc=False),
    )(_sc_gather_body)
    return f(table, idx3, wv3)


def _gcn_body(s, di, h, W, b, g, be, hn_out, hpn_out):
    dii = di[...]
    agg = dii * s[...] + (dii * dii) * h[...]
    out = jnp.dot(agg, W[...], preferred_element_type=jnp.float32) + b[...]
    hn = _gelu(_ln(out + h[...], g[...], be[...]))
    hn_out[...] = hn
    hpn_out[...] = hn * dii


def _gcn(s, di, h, W, b, g, be):
    r1 = lambda a: a.reshape(1, -1)
    bspec = lambda w: pl.BlockSpec((_B, w), lambda i: (i, 0))
    fspec = lambda *sh: pl.BlockSpec(sh, lambda i: tuple(0 for _ in sh))
    return pl.pallas_call(
        _gcn_body,
        grid=(_NP // _B,),
        in_specs=[bspec(_C), bspec(1), bspec(_C),
                  fspec(_C, _C), fspec(1, _C), fspec(1, _C), fspec(1, _C)],
        out_specs=[bspec(_C), bspec(_C)],
        out_shape=[jax.ShapeDtypeStruct((_NP, _C), jnp.float32),
                   jax.ShapeDtypeStruct((_NP, _C), jnp.float32)],
        compiler_params=pltpu.CompilerParams(
            dimension_semantics=("arbitrary",)),
    )(s, di, h, W, r1(b), r1(g), r1(be))


def _head_body(h2, h, tW1a, tW1b, tb1, tg1, tbe1, tW2, tb2, tg2, tbe2,
               mW, mb, lW, lb, mu_out, lv_out):
    t = (jnp.dot(h2[...], tW1a[...], preferred_element_type=jnp.float32)
         + jnp.dot(h[...], tW1b[...], preferred_element_type=jnp.float32)
         + tb1[...])
    t = _gelu(_ln(t, tg1[...], tbe1[...]))
    t = jnp.dot(t, tW2[...], preferred_element_type=jnp.float32) + tb2[...]
    t = _gelu(_ln(t, tg2[...], tbe2[...]))
    m = jnp.dot(t, mW[...], preferred_element_type=jnp.float32) + mb[...]
    mu_out[...] = jax.nn.softplus(5.0 * m) / 5.0
    lv = jnp.dot(t, lW[...], preferred_element_type=jnp.float32) + lb[...]
    lv_out[...] = jnp.clip(lv, -6.0, 4.0)


def _head(h2, h, p):
    r1 = lambda a: a.reshape(1, -1)
    bspec = lambda w: pl.BlockSpec((_B, w), lambda i: (i, 0))
    fspec = lambda *s: pl.BlockSpec(s, lambda i: tuple(0 for _ in s))
    return pl.pallas_call(
        _head_body,
        grid=(_NP // _B,),
        in_specs=[bspec(_C), bspec(_C),
                  fspec(_C, _C), fspec(_C, _C), fspec(1, _C), fspec(1, _C),
                  fspec(1, _C), fspec(_C, _C), fspec(1, _C), fspec(1, _C),
                  fspec(1, _C), fspec(_C, 27), fspec(1, 27),
                  fspec(_C, 27), fspec(1, 27)],
        out_specs=[bspec(27), bspec(27)],
        out_shape=[jax.ShapeDtypeStruct((_NP, 27), jnp.float32),
                   jax.ShapeDtypeStruct((_NP, 27), jnp.float32)],
        compiler_params=pltpu.CompilerParams(
            dimension_semantics=("arbitrary",)),
    )(h2, h, p['tW1'][:_C], p['tW1'][_C:], r1(p['tb1']), r1(p['tg1']),
      r1(p['tbe1']), p['tW2'], r1(p['tb2']), r1(p['tg2']), r1(p['tbe2']),
      p['mW'], r1(p['mb']), p['lW'], r1(p['lb']))


def kernel(x, positions, params):
    p = params
    pos = jnp.pad(positions, ((0, _NP - _N), (0, 0)), constant_values=1e9)
    xp = jnp.pad(x, ((0, _NP - _N), (0, 0), (0, 0)))

    idx, wv, di = _knn(pos)
    h, hp = _embed(xp, di, p)

    idx3 = idx.reshape(_NW, _NCH, _CHUNK)
    wv3 = wv.reshape(_NW, _NCH, _CHUNK)
    s1 = _gather_reduce(hp, idx3, wv3)
    h1, hp1 = _gcn(s1, di, h, p['gW1'], p['gb1'], p['gg1'], p['gbe1'])
    s2 = _gather_reduce(hp1, idx3, wv3)
    h2, _ = _gcn(s2, di, h1, p['gW2'], p['gb2'], p['gg2'], p['gbe2'])

    mu27, lv27 = _head(h2, h, p)
    mu = mu27[:_N].reshape(-1, 3, 3, 3)
    lv = lv27[:_N].reshape(-1, 3, 3, 3)
    return mu, lv


# speedup vs baseline: 3.0909x; 3.0909x over previous
import functools

import jax
import jax.numpy as jnp
from jax import lax
from jax.experimental import pallas as pl
from jax.experimental.pallas import tpu as pltpu
from jax.experimental.pallas import tpu_sc as plsc

_N = 10000
_NP = 10240
_GP = 80
_LN = 128
_KNN = 16
_C = 64
_BIG = 3.0e38


def _ln(x, g, b):
    m = x.mean(-1, keepdims=True)
    v = ((x - m) ** 2).mean(-1, keepdims=True)
    return (x - m) / jnp.sqrt(v + 1e-5) * g + b


def _gelu(x):
    return x * 0.5 * (1.0 + lax.erf(x * (2.0 ** -0.5)))


_R = 512
_NPASS = 3


def _knn_body(pxr, pyr, pzr, pxa, pya, pza, idx_out, wv_out, di_out):
    base = pl.program_id(0) * _R
    px = pxr[...]
    py = pyr[...]
    pz = pzr[...]

    dx = px[:, :, None] - pxa[...][None, :, :]
    dy = py[:, :, None] - pya[...][None, :, :]
    dz = pz[:, :, None] - pza[...][None, :, :]
    d = dx * dx + dy * dy + dz * dz
    g_iota = lax.broadcasted_iota(jnp.int32, (_R, _GP, _LN), 1)
    key = lax.bitcast_convert_type(
        (lax.bitcast_convert_type(d + 1.0, jnp.int32) & -128) | g_iota,
        jnp.float32)

    ms = []
    for j in range(_NPASS):
        m = jnp.min(key, axis=1)
        ms.append(m)
        if j < _NPASS - 1:
            key = jnp.where(key == m[:, None, :], _BIG, key)

    pool = jnp.concatenate(ms, axis=1)
    slot = lax.broadcasted_iota(jnp.int32, (_R, _NPASS * _LN), 1)
    pool_bits = lax.bitcast_convert_type(pool, jnp.int32)
    col = (pool_bits & 127) * _LN + (slot & 127)
    row = base + lax.broadcasted_iota(jnp.int32, (_R, _NPASS * _LN), 0)
    pool = jnp.where(col == row, _BIG, pool)

    idx_cols, wv_cols = [], []
    for _ in range(_KNN):
        m = jnp.min(pool, axis=1, keepdims=True)
        ic = jnp.min(jnp.where(pool == m, col, _NP), axis=1, keepdims=True)
        idx_cols.append(ic)
        dm = lax.bitcast_convert_type(
            lax.bitcast_convert_type(m, jnp.int32) & -128, jnp.float32) - 1.0
        wv_cols.append(jnp.exp(-dm))
        pool = jnp.where(col == ic, _BIG, pool)

    wv = jnp.concatenate(wv_cols, axis=1)
    idx_out[...] = jnp.concatenate(idx_cols, axis=1)
    wv_out[...] = wv
    deg = 1.0 + jnp.sum(wv, axis=1, keepdims=True)
    di_out[...] = jnp.minimum(lax.rsqrt(deg), 1e4)


def _knn(pos):
    pos = pos * jnp.array([[50000000.0 ** -0.5, 50000000.0 ** -0.5,
                            180000.0 ** -0.5]], jnp.float32)
    pxr = pos[:, 0:1]
    pyr = pos[:, 1:2]
    pzr = pos[:, 2:3]
    pxa = pos[:, 0].reshape(_GP, _LN)
    pya = pos[:, 1].reshape(_GP, _LN)
    pza = pos[:, 2].reshape(_GP, _LN)
    rspec = pl.BlockSpec((_R, 1), lambda i: (i, 0))
    aspec = pl.BlockSpec((_GP, _LN), lambda i: (0, 0))
    return pl.pallas_call(
        _knn_body,
        grid=(_NP // _R,),
        in_specs=[rspec, rspec, rspec, aspec, aspec, aspec],
        out_specs=[pl.BlockSpec((_R, _KNN), lambda i: (i, 0)),
                   pl.BlockSpec((_R, _KNN), lambda i: (i, 0)),
                   pl.BlockSpec((_R, 1), lambda i: (i, 0))],
        out_shape=[jax.ShapeDtypeStruct((_NP, _KNN), jnp.int32),
                   jax.ShapeDtypeStruct((_NP, _KNN), jnp.float32),
                   jax.ShapeDtypeStruct((_NP, 1), jnp.float32)],
        compiler_params=pltpu.CompilerParams(
            dimension_semantics=("arbitrary",)),
    )(pxr, pyr, pzr, pxa, pya, pza)


_B = 512


def _embed_body(xs5, xs6, xs7, xs8, xs9, xi5, xi6, xi7, xi8, xi9, di,
                Ws, bs, gs, bes, Wi, bi, gi, bei,
                cwt1, cb1, bng1, bnb1, cwt2, cb2, bng2, bnb2,
                h_out, hp_out):
    xs = [xs5, xs6, xs7, xs8, xs9]
    xi = [xi5, xi6, xi7, xi8, xi9]
    e = []
    for t in range(5):
        es = jnp.dot(xs[t][...], Ws[...], preferred_element_type=jnp.float32)
        ei = jnp.dot(xi[t][...], Wi[...], preferred_element_type=jnp.float32)
        es = _gelu(_ln(es + bs[...], gs[...], bes[...]))
        ei = _gelu(_ln(ei + bi[...], gi[...], bei[...]))
        e.append(es + ei)
    bnscale = 1.0 / jnp.sqrt(1.0 + 1e-5)
    o1 = []
    for t in range(2, 5):
        c = (jnp.dot(e[t - 2][...], cwt1[0], preferred_element_type=jnp.float32)
             + jnp.dot(e[t - 1][...], cwt1[1], preferred_element_type=jnp.float32)
             + jnp.dot(e[t][...], cwt1[2], preferred_element_type=jnp.float32)
             + cb1[...])
        o1.append(_gelu(c * bnscale * bng1[...] + bnb1[...]))
    c2 = (jnp.dot(o1[0], cwt2[0], preferred_element_type=jnp.float32)
          + jnp.dot(o1[1], cwt2[1], preferred_element_type=jnp.float32)
          + jnp.dot(o1[2], cwt2[2], preferred_element_type=jnp.float32)
          + cb2[...])
    o2 = _gelu(c2 * bnscale * bng2[...] + bnb2[...])
    h = o2 + e[4]
    h_out[...] = h
    hp_out[...] = h * di[...]


def _embed(xp, di, p):
    xs = [xp[:, t, :9] for t in range(5, 10)]
    xi = [xp[:, t, 9:] for t in range(5, 10)]
    cwt1 = jnp.transpose(p['cw1'], (2, 1, 0))
    cwt2 = jnp.transpose(p['cw2'], (2, 1, 0))
    r1 = lambda a: a.reshape(1, -1)
    bspec = lambda w: pl.BlockSpec((_B, w), lambda i: (i, 0))
    fspec = lambda *s: pl.BlockSpec(s, lambda i: tuple(0 for _ in s))
    args = (xs + xi
            + [di, p['Ws'], r1(p['bs']), r1(p['gs']), r1(p['bes']),
               p['Wi'], r1(p['bi']), r1(p['gi']), r1(p['bei']),
               cwt1, r1(p['cb1']), r1(p['bng1']), r1(p['bnb1']),
               cwt2, r1(p['cb2']), r1(p['bng2']), r1(p['bnb2'])])
    specs = ([bspec(9)] * 5 + [bspec(19)] * 5
             + [bspec(1), fspec(9, _C), fspec(1, _C), fspec(1, _C), fspec(1, _C),
                fspec(19, _C), fspec(1, _C), fspec(1, _C), fspec(1, _C),
                fspec(3, _C, _C), fspec(1, _C), fspec(1, _C), fspec(1, _C),
                fspec(3, _C, _C), fspec(1, _C), fspec(1, _C), fspec(1, _C)])
    return pl.pallas_call(
        _embed_body,
        grid=(_NP // _B,),
        in_specs=specs,
        out_specs=[bspec(_C), bspec(_C)],
        out_shape=[jax.ShapeDtypeStruct((_NP, _C), jnp.float32),
                   jax.ShapeDtypeStruct((_NP, _C), jnp.float32)],
        compiler_params=pltpu.CompilerParams(
            dimension_semantics=("arbitrary",)),
    )(*args)


_NW = 32
_EP = _NP * _KNN
_BPW = _EP // _NW
_CHUNK = 128
_NCH = _BPW // _CHUNK


_NBUF = 4
_RPC = _CHUNK // _KNN


def _sc_gather_body(table_hbm, idx_hbm, wv_hbm, out_hbm, idx_v, wv_v, acc_v,
                    *bufs_sems):
    bufs = bufs_sems[:_NBUF]
    sems = bufs_sems[_NBUF:]
    wid = lax.axis_index("s") * 2 + lax.axis_index("c")
    rbase = wid * (_BPW // _KNN)
    pltpu.sync_copy(idx_hbm.at[wid], idx_v)
    pltpu.sync_copy(wv_hbm.at[wid], wv_v)
    for b in range(_NBUF):
        pltpu.async_copy(table_hbm.at[idx_v.at[b]], bufs[b], sems[b])

    def _reduce_chunk(j, b):
        def rbody(r, carry):
            wvr = wv_v[j, pl.ds(r * _KNN, _KNN)]
            nq = _C // 16
            accs = [jnp.zeros((16,), jnp.float32)] * nq
            for k in range(_KNN):
                w = wvr[k]
                for q in range(nq):
                    accs[q] = accs[q] + w * bufs[b][r * _KNN + k,
                                                    pl.ds(q * 16, 16)]
            for q in range(nq):
                acc_v[r, pl.ds(q * 16, 16)] = accs[q]
            return carry

        lax.fori_loop(0, _RPC, rbody, 0)
        pltpu.sync_copy(acc_v, out_hbm.at[pl.ds(rbase + j * _RPC, _RPC)])

    def body(i, carry):
        for b in range(_NBUF):
            j = _NBUF * i + b
            pltpu.make_async_copy(
                table_hbm.at[idx_v.at[j]], bufs[b], sems[b]).wait()
            _reduce_chunk(j, b)
            pltpu.async_copy(table_hbm.at[idx_v.at[j + _NBUF]], bufs[b],
                             sems[b])
        return carry

    lax.fori_loop(0, _NCH // _NBUF - 1, body, 0)
    for b in range(_NBUF):
        j = _NCH - _NBUF + b
        pltpu.make_async_copy(
            table_hbm.at[idx_v.at[j]], bufs[b], sems[b]).wait()
        _reduce_chunk(j, b)


def _gather_reduce(table, idx3, wv3):
    mesh = plsc.VectorSubcoreMesh(core_axis_name="c", subcore_axis_name="s")
    f = functools.partial(
        pl.kernel,
        mesh=mesh,
        out_type=jax.ShapeDtypeStruct((_NP, _C), jnp.float32),
        scratch_types=(
            [pltpu.VMEM((_NCH, _CHUNK), jnp.int32),
             pltpu.VMEM((_NCH, _CHUNK), jnp.float32),
             pltpu.VMEM((_RPC, _C), jnp.float32)]
            + [pltpu.VMEM((_CHUNK, _C), jnp.float32)] * _NBUF
            + [pltpu.SemaphoreType.DMA] * _NBUF
        ),
        compiler_params=pltpu.CompilerParams(use_tc_tiling_on_sc=False),
    )(_sc_gather_body)
    return f(table, idx3, wv3)


def _gcn_body(s, di, h, W, b, g, be, hn_out, hpn_out):
    dii = di[...]
    agg = dii * s[...] + (dii * dii) * h[...]
    out = jnp.dot(agg, W[...], preferred_element_type=jnp.float32) + b[...]
    hn = _gelu(_ln(out + h[...], g[...], be[...]))
    hn_out[...] = hn
    hpn_out[...] = hn * dii


def _gcn(s, di, h, W, b, g, be):
    r1 = lambda a: a.reshape(1, -1)
    bspec = lambda w: pl.BlockSpec((_B, w), lambda i: (i, 0))
    fspec = lambda *sh: pl.BlockSpec(sh, lambda i: tuple(0 for _ in sh))
    return pl.pallas_call(
        _gcn_body,
        grid=(_NP // _B,),
        in_specs=[bspec(_C), bspec(1), bspec(_C),
                  fspec(_C, _C), fspec(1, _C), fspec(1, _C), fspec(1, _C)],
        out_specs=[bspec(_C), bspec(_C)],
        out_shape=[jax.ShapeDtypeStruct((_NP, _C), jnp.float32),
                   jax.ShapeDtypeStruct((_NP, _C), jnp.float32)],
        compiler_params=pltpu.CompilerParams(
            dimension_semantics=("arbitrary",)),
    )(s, di, h, W, r1(b), r1(g), r1(be))


def _head_body(h2, h, tW1a, tW1b, tb1, tg1, tbe1, tW2, tb2, tg2, tbe2,
               mW, mb, lW, lb, mu_out, lv_out):
    t = (jnp.dot(h2[...], tW1a[...], preferred_element_type=jnp.float32)
         + jnp.dot(h[...], tW1b[...], preferred_element_type=jnp.float32)
         + tb1[...])
    t = _gelu(_ln(t, tg1[...], tbe1[...]))
    t = jnp.dot(t, tW2[...], preferred_element_type=jnp.float32) + tb2[...]
    t = _gelu(_ln(t, tg2[...], tbe2[...]))
    m = jnp.dot(t, mW[...], preferred_element_type=jnp.float32) + mb[...]
    mu_out[...] = jax.nn.softplus(5.0 * m) / 5.0
    lv = jnp.dot(t, lW[...], preferred_element_type=jnp.float32) + lb[...]
    lv_out[...] = jnp.clip(lv, -6.0, 4.0)


def _head(h2, h, p):
    r1 = lambda a: a.reshape(1, -1)
    bspec = lambda w: pl.BlockSpec((_B, w), lambda i: (i, 0))
    fspec = lambda *s: pl.BlockSpec(s, lambda i: tuple(0 for _ in s))
    return pl.pallas_call(
        _head_body,
        grid=(_NP // _B,),
        in_specs=[bspec(_C), bspec(_C),
                  fspec(_C, _C), fspec(_C, _C), fspec(1, _C), fspec(1, _C),
                  fspec(1, _C), fspec(_C, _C), fspec(1, _C), fspec(1, _C),
                  fspec(1, _C), fspec(_C, 27), fspec(1, 27),
                  fspec(_C, 27), fspec(1, 27)],
        out_specs=[bspec(27), bspec(27)],
        out_shape=[jax.ShapeDtypeStruct((_NP, 27), jnp.float32),
                   jax.ShapeDtypeStruct((_NP, 27), jnp.float32)],
        compiler_params=pltpu.CompilerParams(
            dimension_semantics=("arbitrary",)),
    )(h2, h, p['tW1'][:_C], p['tW1'][_C:], r1(p['tb1']), r1(p['tg1']),
      r1(p['tbe1']), p['tW2'], r1(p['tb2']), r1(p['tg2']), r1(p['tbe2']),
      p['mW'], r1(p['mb']), p['lW'], r1(p['lb']))


def kernel(x, positions, params):
    p = params
    pos = jnp.pad(positions, ((0, _NP - _N), (0, 0)), constant_values=1e9)
    xp = jnp.pad(x, ((0, _NP - _N), (0, 0), (0, 0)))

    idx, wv, di = _knn(pos)
    h, hp = _embed(xp, di, p)

    idx3 = idx.reshape(_NW, _NCH, _CHUNK)
    wv3 = wv.reshape(_NW, _NCH, _CHUNK)
    s1 = _gather_reduce(hp, idx3, wv3)
    h1, hp1 = _gcn(s1, di, h, p['gW1'], p['gb1'], p['gg1'], p['gbe1'])
    s2 = _gather_reduce(hp1, idx3, wv3)
    h2, _ = _gcn(s2, di, h1, p['gW2'], p['gb2'], p['gg2'], p['gbe2'])

    mu27, lv27 = _head(h2, h, p)
    mu = mu27[:_N].reshape(-1, 3, 3, 3)
    lv = lv27[:_N].reshape(-1, 3, 3, 3)
    return mu, lv
